# Initial kernel scaffold; baseline (speedup 1.0000x reference)
import functools

import jax
import jax.numpy as jnp
from jax import lax
from jax.experimental import pallas as pl
from jax.experimental.pallas import tpu as pltpu
from jax.experimental.pallas import tpu_sc as plsc

N = 10000
E = 320000
DN = 128
DH = 256
DC = 64
H = 128
NS = 16
RPT = N // NS
ECH = 80

EPT_H = E // NS
NCH_H = EPT_H // ECH
EPT_S = E // (2 * NS)
NCH_S = EPT_S // ECH

_mesh = plsc.VectorSubcoreMesh(core_axis_name="c", subcore_axis_name="s")

_RB = 2000


@functools.partial(
    pl.kernel,
    out_type=jax.ShapeDtypeStruct((2 * N, 16), jnp.float32),
    mesh=_mesh,
    scratch_types=[
        pltpu.VMEM((NCH_S, ECH), jnp.int32),
        pltpu.VMEM((ECH, 16), jnp.float32),
        pltpu.VMEM((RPT, 16), jnp.float32),
        pltpu.VMEM_SHARED((N, 16), jnp.float32),
    ],
)
def _deg_kernel(dst_hbm, out_hbm, dst_v, ones_v, zero_v, acc):
    c = lax.axis_index("c")
    s = lax.axis_index("s")

    @pl.loop(0, ECH)
    def _(i):
        ones_v[i, :] = jnp.full((16,), 1.0, jnp.float32)

    @pl.loop(0, RPT)
    def _(i):
        zero_v[i, :] = jnp.zeros((16,), jnp.float32)

    r0 = s * RPT
    pltpu.sync_copy(zero_v, acc.at[pl.ds(r0, RPT)])
    pltpu.sync_copy(dst_hbm.at[c, s], dst_v)
    plsc.subcore_barrier()

    @pl.loop(0, NCH_S)
    def _(k):
        pltpu.sync_copy(ones_v, acc.at[dst_v.at[k]], add=True)

    plsc.subcore_barrier()
    pltpu.sync_copy(acc.at[pl.ds(r0, RPT)], out_hbm.at[pl.ds(c * N + r0, RPT)])


def _make_prop(width, nch):
    @functools.partial(
        pl.kernel,
        out_type=jax.ShapeDtypeStruct((2 * N, width), jnp.float32),
        mesh=_mesh,
        scratch_types=[
            pltpu.VMEM((nch, ECH), jnp.int32),
            pltpu.VMEM((nch, ECH), jnp.int32),
            pltpu.VMEM((ECH, width), jnp.float32),
            pltpu.VMEM((ECH, width), jnp.float32),
            pltpu.VMEM_SHARED((N, width), jnp.float32),
            pltpu.SemaphoreType.DMA,
            pltpu.SemaphoreType.DMA,
        ],
    )
    def _prop(t_hbm, src_hbm, dst_hbm, out_hbm, src_v, dst_v, rows_a, rows_b,
              acc, sem_a, sem_b):
        c = lax.axis_index("c")
        s = lax.axis_index("s")
        r0 = s * RPT

        pltpu.sync_copy(t_hbm.at[pl.ds(c * N + r0, RPT)], acc.at[pl.ds(r0, RPT)])
        pltpu.sync_copy(src_hbm.at[c, s], src_v)
        pltpu.sync_copy(dst_hbm.at[c, s], dst_v)
        plsc.subcore_barrier()

        def _start(k, buf, sem):
            pltpu.make_async_copy(t_hbm.at[src_v.at[k]], buf, sem).start()

        def _wait(buf, sem):
            pltpu.make_async_copy(t_hbm.at[pl.ds(0, ECH)], buf, sem).wait()

        _start(0, rows_a, sem_a)

        @pl.loop(0, nch // 2)
        def _(j):
            k = j * 2
            _wait(rows_a, sem_a)
            _start(k + 1, rows_b, sem_b)
            pltpu.sync_copy(rows_a, acc.at[dst_v.at[k]], add=True)
            _wait(rows_b, sem_b)

            @pl.when(j < nch // 2 - 1)
            def _():
                _start(k + 2, rows_a, sem_a)

            pltpu.sync_copy(rows_b, acc.at[dst_v.at[k + 1]], add=True)

        plsc.subcore_barrier()
        pltpu.sync_copy(acc.at[pl.ds(r0, RPT)], out_hbm.at[pl.ds(c * N + r0, RPT)])

    return _prop


_prop_half = _make_prop(H, NCH_H)
_prop_full = _make_prop(DC, NCH_S)


def _mm0(x, W0):
    def body(x_ref, w_ref, o_ref):
        p = jnp.dot(x_ref[...], w_ref[...], preferred_element_type=jnp.float32)
        o_ref[0] = p[:, :H]
        o_ref[1] = p[:, H:]

    return pl.pallas_call(
        body,
        grid=(N // _RB,),
        in_specs=[pl.BlockSpec((_RB, DN), lambda i: (i, 0)),
                  pl.BlockSpec((DN, DH), lambda i: (0, 0))],
        out_specs=pl.BlockSpec((2, _RB, H), lambda i: (0, i, 0)),
        out_shape=jax.ShapeDtypeStruct((2, N, H), jnp.float32),
    )(x, W0)


def _finish_t0(degp, hw0):
    def body(d_ref, h_ref, dinv_ref, t_ref):
        deg = d_ref[0, :, 0:1] + d_ref[1, :, 0:1] + 1.0
        dv = lax.rsqrt(jnp.maximum(deg, 1e-12))
        dinv_ref[...] = dv
        t_ref[...] = h_ref[...] * dv[None]

    return pl.pallas_call(
        body,
        grid=(N // _RB,),
        in_specs=[pl.BlockSpec((2, _RB, 16), lambda i: (0, i, 0)),
                  pl.BlockSpec((2, _RB, H), lambda i: (0, i, 0))],
        out_specs=[pl.BlockSpec((_RB, 1), lambda i: (i, 0)),
                   pl.BlockSpec((2, _RB, H), lambda i: (0, i, 0))],
        out_shape=[jax.ShapeDtypeStruct((N, 1), jnp.float32),
                   jax.ShapeDtypeStruct((2, N, H), jnp.float32)],
    )(degp, hw0)


def _layer_mid(s, dinv, b, W):
    def body(s_ref, d_ref, b_ref, w_ref, o_ref):
        dv = d_ref[...]
        hcat = jnp.concatenate([s_ref[0], s_ref[1]], axis=1)
        h = jnp.maximum(hcat * dv + b_ref[...], 0.0)
        p = jnp.dot(h, w_ref[...], preferred_element_type=jnp.float32) * dv
        o_ref[0] = p[:, :H]
        o_ref[1] = p[:, H:]

    return pl.pallas_call(
        body,
        grid=(N // _RB,),
        in_specs=[pl.BlockSpec((2, _RB, H), lambda i: (0, i, 0)),
                  pl.BlockSpec((_RB, 1), lambda i: (i, 0)),
                  pl.BlockSpec((1, DH), lambda i: (0, 0)),
                  pl.BlockSpec((DH, DH), lambda i: (0, 0))],
        out_specs=pl.BlockSpec((2, _RB, H), lambda i: (0, i, 0)),
        out_shape=jax.ShapeDtypeStruct((2, N, H), jnp.float32),
    )(s, dinv, b, W)


def _layer_last(s, dinv, b, W):
    def body(s_ref, d_ref, b_ref, w_ref, o_ref):
        dv = d_ref[...]
        hcat = jnp.concatenate([s_ref[0], s_ref[1]], axis=1)
        h = jnp.maximum(hcat * dv + b_ref[...], 0.0)
        o_ref[...] = jnp.dot(h, w_ref[...], preferred_element_type=jnp.float32) * dv

    return pl.pallas_call(
        body,
        grid=(N // _RB,),
        in_specs=[pl.BlockSpec((2, _RB, H), lambda i: (0, i, 0)),
                  pl.BlockSpec((_RB, 1), lambda i: (i, 0)),
                  pl.BlockSpec((1, DC), lambda i: (0, 0)),
                  pl.BlockSpec((DH, DC), lambda i: (0, 0))],
        out_specs=pl.BlockSpec((_RB, DC), lambda i: (i, 0)),
        out_shape=jax.ShapeDtypeStruct((N, DC), jnp.float32),
    )(s, dinv, b, W)


def _combine_out(s2, dinv, bh):
    def body(s_ref, d_ref, b_ref, o_ref):
        o_ref[...] = (s_ref[0] + s_ref[1]) * d_ref[...] + b_ref[...]

    return pl.pallas_call(
        body,
        grid=(N // _RB,),
        in_specs=[pl.BlockSpec((2, _RB, DC), lambda i: (0, i, 0)),
                  pl.BlockSpec((_RB, 1), lambda i: (i, 0)),
                  pl.BlockSpec((1, DC), lambda i: (0, 0))],
        out_specs=pl.BlockSpec((_RB, DC), lambda i: (i, 0)),
        out_shape=jax.ShapeDtypeStruct((N, DC), jnp.float32),
    )(s2, dinv, bh)


def kernel(x, edge_index, W0, b0, W1, b1, Wh, bh):
    src = edge_index[0]
    dst = edge_index[1]

    src2 = jnp.concatenate([src, src + N]).reshape(2, NS, NCH_H, ECH)
    dst2 = jnp.broadcast_to(dst.reshape(1, NS, NCH_H, ECH), (2, NS, NCH_H, ECH))
    src_sp = src.reshape(2, NS, NCH_S, ECH)
    dst_sp = dst.reshape(2, NS, NCH_S, ECH)

    degp = _deg_kernel(dst_sp)
    hw0 = _mm0(x, W0)
    dinv, t0 = _finish_t0(degp.reshape(2, N, 16), hw0)
    s0 = _prop_half(t0.reshape(2 * N, H), src2, dst2)
    t1 = _layer_mid(s0.reshape(2, N, H), dinv, b0.reshape(1, DH), W1)
    s1 = _prop_half(t1.reshape(2 * N, H), src2, dst2)
    t2 = _layer_last(s1.reshape(2, N, H), dinv, b1.reshape(1, DH), Wh)
    t2z = jnp.concatenate([t2, jnp.zeros((N, DC), jnp.float32)], axis=0)
    s2 = _prop_full(t2z, src_sp, dst_sp)
    out = _combine_out(s2.reshape(2, N, DC), dinv, bh.reshape(1, DC))
    return out


# speedup vs baseline: 6.3690x; 6.3690x over previous
import functools

import jax
import jax.numpy as jnp
from jax import lax
from jax.experimental import pallas as pl
from jax.experimental.pallas import tpu as pltpu
from jax.experimental.pallas import tpu_sc as plsc

N = 10000
E = 320000
DN = 128
DH = 256
DC = 64
H = 128
NS = 16

ECH = 40
SUP = 50
NSUP_H = E // NS // (SUP * ECH)
NSUP_S = E // (2 * NS) // (SUP * ECH)

RCH = 40
NRCH = N // RCH
CPT = 16
NRPAD = CPT * NS

_mesh = plsc.VectorSubcoreMesh(core_axis_name="c", subcore_axis_name="s")

_RB = 2000


def _row_chunks(s, body):
    @pl.loop(0, CPT)
    def _(k):
        chunk = s * CPT + k

        @pl.when(chunk < NRCH)
        def _():
            body(k, chunk)


@functools.partial(
    pl.kernel,
    out_type=jax.ShapeDtypeStruct((2 * N, H), jnp.float32),
    mesh=_mesh,
    scratch_types=[
        pltpu.VMEM((ECH,), jnp.int32),
        pltpu.VMEM((RCH,), jnp.int32),
        pltpu.VMEM((ECH, H), jnp.float32),
        pltpu.VMEM((RCH, H), jnp.float32),
        pltpu.VMEM_SHARED((N, H), jnp.float32),
    ],
)
def _deg_kernel(dst_hbm, ids_hbm, ones_hbm, zeros_hbm, out_hbm,
                dst_v, ids_v, ones_v, stage_v, acc):
    c = lax.axis_index("c")
    s = lax.axis_index("s")

    pltpu.sync_copy(ones_hbm, ones_v)
    pltpu.sync_copy(zeros_hbm, stage_v)

    def _init(k, chunk):
        pltpu.sync_copy(ids_hbm.at[pl.ds(chunk * RCH, RCH)], ids_v)
        pltpu.sync_copy(stage_v, acc.at[ids_v])

    _row_chunks(s, _init)
    plsc.subcore_barrier()

    ebase = c * (E // 2) + s * (E // 2 // NS)

    @pl.loop(0, E // 2 // NS // ECH)
    def _(k):
        pltpu.sync_copy(dst_hbm.at[pl.ds(ebase + k * ECH, ECH)], dst_v)
        pltpu.sync_copy(ones_v, acc.at[dst_v], add=True)

    plsc.subcore_barrier()

    def _out(k, chunk):
        pltpu.sync_copy(ids_hbm.at[pl.ds(chunk * RCH, RCH)], ids_v)
        pltpu.sync_copy(acc.at[ids_v], stage_v)
        pltpu.sync_copy(stage_v, out_hbm.at[pl.ds(c * N + chunk * RCH, RCH)])

    _row_chunks(s, _out)


def _make_prop(nsup, split_by_core):
    @functools.partial(
        pl.kernel,
        out_type=jax.ShapeDtypeStruct((2 * N, H), jnp.float32),
        mesh=_mesh,
        scratch_types=[
            pltpu.VMEM((SUP, ECH), jnp.int32),
            pltpu.VMEM((ECH,), jnp.int32),
            pltpu.VMEM((RCH,), jnp.int32),
            pltpu.VMEM((ECH, H), jnp.float32),
            pltpu.VMEM((ECH, H), jnp.float32),
            pltpu.VMEM((RCH, H), jnp.float32),
            pltpu.VMEM_SHARED((N, H), jnp.float32),
            pltpu.SemaphoreType.DMA,
            pltpu.SemaphoreType.DMA,
        ],
    )
    def _prop(t_hbm, src_hbm, dst_hbm, ids_hbm, zeros_hbm, out_hbm,
              src_v, dst_v, ids_v, rows_a, rows_b, stage_v, acc,
              sem_a, sem_b):
        c = lax.axis_index("c")
        s = lax.axis_index("s")

        pltpu.sync_copy(zeros_hbm, stage_v)

        def _init(k, chunk):
            pltpu.sync_copy(ids_hbm.at[pl.ds(chunk * RCH, RCH)], ids_v)
            pltpu.sync_copy(stage_v, acc.at[ids_v])

        _row_chunks(s, _init)
        plsc.subcore_barrier()

        if split_by_core:
            ebase = c * (E // 2) + s * (E // 2 // NS)
        else:
            ebase = s * (E // NS)

        @pl.loop(0, nsup)
        def _(u):
            pltpu.sync_copy(src_hbm.at[c, s, u], src_v)
            sbase = ebase + u * SUP * ECH

            @pl.loop(0, SUP)
            def _(k):
                pltpu.sync_copy(t_hbm.at[src_v.at[k]], rows_a)
                pltpu.sync_copy(dst_hbm.at[pl.ds(sbase + k * ECH, ECH)], dst_v)
                pltpu.sync_copy(rows_a, acc.at[dst_v], add=True)

        plsc.subcore_barrier()

        def _out(k, chunk):
            pltpu.sync_copy(ids_hbm.at[pl.ds(chunk * RCH, RCH)], ids_v)
            pltpu.sync_copy(acc.at[ids_v], stage_v)
            pltpu.sync_copy(stage_v,
                            out_hbm.at[pl.ds(c * N + chunk * RCH, RCH)])

        _row_chunks(s, _out)

    return _prop


_prop_half = _make_prop(NSUP_H, False)
_prop_full = _make_prop(NSUP_S, True)


def _mm0(x, W0):
    def body(x_ref, w_ref, o_ref):
        p = jnp.dot(x_ref[...], w_ref[...], preferred_element_type=jnp.float32)
        o_ref[0] = p[:, :H]
        o_ref[1] = p[:, H:]

    return pl.pallas_call(
        body,
        grid=(N // _RB,),
        in_specs=[pl.BlockSpec((_RB, DN), lambda i: (i, 0)),
                  pl.BlockSpec((DN, DH), lambda i: (0, 0))],
        out_specs=pl.BlockSpec((2, _RB, H), lambda i: (0, i, 0)),
        out_shape=jax.ShapeDtypeStruct((2, N, H), jnp.float32),
    )(x, W0)


def _finish_t0(degp, hw0):
    def body(d_ref, h_ref, dinv_ref, t_ref):
        deg = d_ref[0, :, 0:1] + d_ref[1, :, 0:1] + 1.0
        dv = lax.rsqrt(jnp.maximum(deg, 1e-12))
        dinv_ref[...] = dv
        t_ref[...] = h_ref[...] * dv[None]

    return pl.pallas_call(
        body,
        grid=(N // _RB,),
        in_specs=[pl.BlockSpec((2, _RB, H), lambda i: (0, i, 0)),
                  pl.BlockSpec((2, _RB, H), lambda i: (0, i, 0))],
        out_specs=[pl.BlockSpec((_RB, 1), lambda i: (i, 0)),
                   pl.BlockSpec((2, _RB, H), lambda i: (0, i, 0))],
        out_shape=[jax.ShapeDtypeStruct((N, 1), jnp.float32),
                   jax.ShapeDtypeStruct((2, N, H), jnp.float32)],
    )(degp, hw0)


def _layer_mid(sm, t, dinv, b, W):
    def body(s_ref, t_ref, d_ref, b_ref, w_ref, o_ref):
        dv = d_ref[...]
        hcat = jnp.concatenate([s_ref[0] + t_ref[0], s_ref[1] + t_ref[1]],
                               axis=1)
        h = jnp.maximum(hcat * dv + b_ref[...], 0.0)
        p = jnp.dot(h, w_ref[...], preferred_element_type=jnp.float32) * dv
        o_ref[0] = p[:, :H]
        o_ref[1] = p[:, H:]

    return pl.pallas_call(
        body,
        grid=(N // _RB,),
        in_specs=[pl.BlockSpec((2, _RB, H), lambda i: (0, i, 0)),
                  pl.BlockSpec((2, _RB, H), lambda i: (0, i, 0)),
                  pl.BlockSpec((_RB, 1), lambda i: (i, 0)),
                  pl.BlockSpec((1, DH), lambda i: (0, 0)),
                  pl.BlockSpec((DH, DH), lambda i: (0, 0))],
        out_specs=pl.BlockSpec((2, _RB, H), lambda i: (0, i, 0)),
        out_shape=jax.ShapeDtypeStruct((2, N, H), jnp.float32),
    )(sm, t, dinv, b, W)


def _layer_last(sm, t, dinv, b, W):
    def body(s_ref, t_ref, d_ref, b_ref, w_ref, o_ref):
        dv = d_ref[...]
        hcat = jnp.concatenate([s_ref[0] + t_ref[0], s_ref[1] + t_ref[1]],
                               axis=1)
        h = jnp.maximum(hcat * dv + b_ref[...], 0.0)
        p = jnp.dot(h, w_ref[...], preferred_element_type=jnp.float32) * dv
        o_ref[...] = jnp.pad(p, ((0, 0), (0, H - DC)))

    return pl.pallas_call(
        body,
        grid=(N // _RB,),
        in_specs=[pl.BlockSpec((2, _RB, H), lambda i: (0, i, 0)),
                  pl.BlockSpec((2, _RB, H), lambda i: (0, i, 0)),
                  pl.BlockSpec((_RB, 1), lambda i: (i, 0)),
                  pl.BlockSpec((1, DH), lambda i: (0, 0)),
                  pl.BlockSpec((DH, DC), lambda i: (0, 0))],
        out_specs=pl.BlockSpec((_RB, H), lambda i: (i, 0)),
        out_shape=jax.ShapeDtypeStruct((N, H), jnp.float32),
    )(sm, t, dinv, b, W)


def _combine_out(s2, t2, dinv, bh):
    def body(s_ref, t_ref, d_ref, b_ref, o_ref):
        tot = s_ref[0, :, :DC] + s_ref[1, :, :DC] + t_ref[:, :DC]
        o_ref[...] = tot * d_ref[...] + b_ref[...]

    return pl.pallas_call(
        body,
        grid=(N // _RB,),
        in_specs=[pl.BlockSpec((2, _RB, H), lambda i: (0, i, 0)),
                  pl.BlockSpec((_RB, H), lambda i: (i, 0)),
                  pl.BlockSpec((_RB, 1), lambda i: (i, 0)),
                  pl.BlockSpec((1, DC), lambda i: (0, 0))],
        out_specs=pl.BlockSpec((_RB, DC), lambda i: (i, 0)),
        out_shape=jax.ShapeDtypeStruct((N, DC), jnp.float32),
    )(s2, t2, dinv, bh)


def kernel(x, edge_index, W0, b0, W1, b1, Wh, bh):
    src = edge_index[0]
    dst = edge_index[1]

    src2 = jnp.concatenate([src, src + N]).reshape(2, NS, NSUP_H, SUP, ECH)
    src_sp = src.reshape(2, NS, NSUP_S, SUP, ECH)
    ids = jnp.arange(N, dtype=jnp.int32)

    ones_c = jnp.ones((ECH, H), jnp.float32)
    zerosH = jnp.zeros((RCH, H), jnp.float32)

    degp = _deg_kernel(dst, ids, ones_c, zerosH)
    hw0 = _mm0(x, W0)
    dinv, t0 = _finish_t0(degp.reshape(2, N, H), hw0)
    t0f = t0.reshape(2 * N, H)
    s0 = _prop_half(t0f, src2, dst, ids, zerosH)
    t1 = _layer_mid(s0.reshape(2, N, H), t0, dinv, b0.reshape(1, DH), W1)
    t1f = t1.reshape(2 * N, H)
    s1 = _prop_half(t1f, src2, dst, ids, zerosH)
    t2 = _layer_last(s1.reshape(2, N, H), t1, dinv, b1.reshape(1, DH), Wh)
    s2 = _prop_full(t2, src_sp, dst, ids, zerosH)
    out = _combine_out(s2.reshape(2, N, H), t2, dinv, bh.reshape(1, DC))
    return out


# speedup vs baseline: 13.7668x; 2.1615x over previous
import functools

import jax
import jax.numpy as jnp
from jax import lax
from jax.experimental import pallas as pl
from jax.experimental.pallas import tpu as pltpu
from jax.experimental.pallas import tpu_sc as plsc

N = 10000
E = 320000
DN = 128
DH = 256
DC = 64
H = 128
NS = 16

ECH = 40
SUP = 50
NSUP_H = E // NS // (SUP * ECH)
NSUP_S = E // (2 * NS) // (SUP * ECH)

RCH = 40
NRCH = N // RCH
CPT = 16
NRPAD = CPT * NS

_mesh = plsc.VectorSubcoreMesh(core_axis_name="c", subcore_axis_name="s")

_RB = 2000


def _row_chunks(s, body):
    @pl.loop(0, CPT)
    def _(k):
        chunk = s * CPT + k

        @pl.when(chunk < NRCH)
        def _():
            body(k, chunk)


@functools.partial(
    pl.kernel,
    out_type=jax.ShapeDtypeStruct((2 * N, H), jnp.float32),
    mesh=_mesh,
    scratch_types=[
        pltpu.VMEM((ECH,), jnp.int32),
        pltpu.VMEM((ECH,), jnp.int32),
        pltpu.VMEM((RCH,), jnp.int32),
        pltpu.VMEM((ECH, H), jnp.float32),
        pltpu.VMEM((RCH, H), jnp.float32),
        pltpu.VMEM_SHARED((N, H), jnp.float32),
        pltpu.SemaphoreType.DMA,
        pltpu.SemaphoreType.DMA,
    ],
)
def _deg_kernel(dst_hbm, ids_hbm, ones_hbm, zeros_hbm, out_hbm,
                dst_va, dst_vb, ids_v, ones_v, stage_v, acc,
                sem_ia, sem_ib):
    c = lax.axis_index("c")
    s = lax.axis_index("s")

    pltpu.sync_copy(ones_hbm, ones_v)
    pltpu.sync_copy(zeros_hbm, stage_v)

    def _init(k, chunk):
        pltpu.sync_copy(ids_hbm.at[pl.ds(chunk * RCH, RCH)], ids_v)
        pltpu.sync_copy(stage_v, acc.at[ids_v])

    _row_chunks(s, _init)
    plsc.subcore_barrier()

    ebase = c * (E // 2) + s * (E // 2 // NS)
    nch = E // 2 // NS // ECH

    def _idx(k, buf, sem):
        return pltpu.make_async_copy(
            dst_hbm.at[pl.ds(ebase + k * ECH, ECH)], buf, sem)

    pltpu.sync_copy(dst_hbm.at[pl.ds(ebase, ECH)], dst_va)

    @pl.loop(0, nch // 2)
    def _(j):
        k = j * 2
        _idx(k + 1, dst_vb, sem_ib).start()
        pltpu.sync_copy(ones_v, acc.at[dst_va], add=True)

        @pl.when(j < nch // 2 - 1)
        def _():
            _idx(k + 2, dst_va, sem_ia).start()

        _idx(k + 1, dst_vb, sem_ib).wait()
        pltpu.sync_copy(ones_v, acc.at[dst_vb], add=True)

        @pl.when(j < nch // 2 - 1)
        def _():
            _idx(k + 2, dst_va, sem_ia).wait()

    plsc.subcore_barrier()

    def _out(k, chunk):
        pltpu.sync_copy(ids_hbm.at[pl.ds(chunk * RCH, RCH)], ids_v)
        pltpu.sync_copy(acc.at[ids_v], stage_v)
        pltpu.sync_copy(stage_v, out_hbm.at[pl.ds(c * N + chunk * RCH, RCH)])

    _row_chunks(s, _out)


def _make_prop(nsup, split_by_core):
    @functools.partial(
        pl.kernel,
        out_type=jax.ShapeDtypeStruct((2 * N, H), jnp.float32),
        mesh=_mesh,
        scratch_types=[
            pltpu.VMEM((SUP, ECH), jnp.int32),
            pltpu.VMEM((ECH,), jnp.int32),
            pltpu.VMEM((ECH,), jnp.int32),
            pltpu.VMEM((RCH,), jnp.int32),
            pltpu.VMEM((ECH, H), jnp.float32),
            pltpu.VMEM((ECH, H), jnp.float32),
            pltpu.VMEM((RCH, H), jnp.float32),
            pltpu.VMEM_SHARED((N, H), jnp.float32),
            pltpu.SemaphoreType.DMA,
            pltpu.SemaphoreType.DMA,
            pltpu.SemaphoreType.DMA,
            pltpu.SemaphoreType.DMA,
        ],
    )
    def _prop(t_hbm, src_hbm, dst_hbm, ids_hbm, zeros_hbm, out_hbm,
              src_v, dst_va, dst_vb, ids_v, rows_a, rows_b, stage_v, acc,
              sem_a, sem_b, sem_ia, sem_ib):
        c = lax.axis_index("c")
        s = lax.axis_index("s")

        pltpu.sync_copy(zeros_hbm, stage_v)

        def _init(k, chunk):
            pltpu.sync_copy(ids_hbm.at[pl.ds(chunk * RCH, RCH)], ids_v)
            pltpu.sync_copy(stage_v, acc.at[ids_v])

        _row_chunks(s, _init)
        plsc.subcore_barrier()

        if split_by_core:
            ebase = c * (E // 2) + s * (E // 2 // NS)
        else:
            ebase = s * (E // NS)

        @pl.loop(0, nsup)
        def _(u):
            pltpu.sync_copy(src_hbm.at[c, s, u], src_v)
            sbase = ebase + u * SUP * ECH

            def _gat(k, buf, sem):
                return pltpu.make_async_copy(t_hbm.at[src_v.at[k]], buf, sem)

            def _idx(k, buf, sem):
                return pltpu.make_async_copy(
                    dst_hbm.at[pl.ds(sbase + k * ECH, ECH)], buf, sem)

            pltpu.sync_copy(dst_hbm.at[pl.ds(sbase, ECH)], dst_va)
            _gat(0, rows_a, sem_a).start()

            @pl.loop(0, SUP // 2)
            def _(j):
                k = j * 2
                _gat(k + 1, rows_b, sem_b).start()
                _idx(k + 1, dst_vb, sem_ib).start()
                _gat(k, rows_a, sem_a).wait()
                pltpu.sync_copy(rows_a, acc.at[dst_va], add=True)

                @pl.when(j < SUP // 2 - 1)
                def _():
                    _gat(k + 2, rows_a, sem_a).start()
                    _idx(k + 2, dst_va, sem_ia).start()

                _gat(k + 1, rows_b, sem_b).wait()
                _idx(k + 1, dst_vb, sem_ib).wait()
                pltpu.sync_copy(rows_b, acc.at[dst_vb], add=True)

                @pl.when(j < SUP // 2 - 1)
                def _():
                    _idx(k + 2, dst_va, sem_ia).wait()

        plsc.subcore_barrier()

        def _out(k, chunk):
            pltpu.sync_copy(ids_hbm.at[pl.ds(chunk * RCH, RCH)], ids_v)
            pltpu.sync_copy(acc.at[ids_v], stage_v)
            pltpu.sync_copy(stage_v,
                            out_hbm.at[pl.ds(c * N + chunk * RCH, RCH)])

        _row_chunks(s, _out)

    return _prop


_prop_half = _make_prop(NSUP_H, False)
_prop_full = _make_prop(NSUP_S, True)


def _mm0(x, W0):
    def body(x_ref, w_ref, o_ref):
        p = jnp.dot(x_ref[...], w_ref[...], preferred_element_type=jnp.float32)
        o_ref[0] = p[:, :H]
        o_ref[1] = p[:, H:]

    return pl.pallas_call(
        body,
        grid=(N // _RB,),
        in_specs=[pl.BlockSpec((_RB, DN), lambda i: (i, 0)),
                  pl.BlockSpec((DN, DH), lambda i: (0, 0))],
        out_specs=pl.BlockSpec((2, _RB, H), lambda i: (0, i, 0)),
        out_shape=jax.ShapeDtypeStruct((2, N, H), jnp.float32),
    )(x, W0)


def _finish_t0(degp, hw0):
    def body(d_ref, h_ref, dinv_ref, t_ref):
        deg = d_ref[0, :, 0:1] + d_ref[1, :, 0:1] + 1.0
        dv = lax.rsqrt(jnp.maximum(deg, 1e-12))
        dinv_ref[...] = dv
        t_ref[...] = h_ref[...] * dv[None]

    return pl.pallas_call(
        body,
        grid=(N // _RB,),
        in_specs=[pl.BlockSpec((2, _RB, H), lambda i: (0, i, 0)),
                  pl.BlockSpec((2, _RB, H), lambda i: (0, i, 0))],
        out_specs=[pl.BlockSpec((_RB, 1), lambda i: (i, 0)),
                   pl.BlockSpec((2, _RB, H), lambda i: (0, i, 0))],
        out_shape=[jax.ShapeDtypeStruct((N, 1), jnp.float32),
                   jax.ShapeDtypeStruct((2, N, H), jnp.float32)],
    )(degp, hw0)


def _layer_mid(sm, t, dinv, b, W):
    def body(s_ref, t_ref, d_ref, b_ref, w_ref, o_ref):
        dv = d_ref[...]
        hcat = jnp.concatenate([s_ref[0] + t_ref[0], s_ref[1] + t_ref[1]],
                               axis=1)
        h = jnp.maximum(hcat * dv + b_ref[...], 0.0)
        p = jnp.dot(h, w_ref[...], preferred_element_type=jnp.float32) * dv
        o_ref[0] = p[:, :H]
        o_ref[1] = p[:, H:]

    return pl.pallas_call(
        body,
        grid=(N // _RB,),
        in_specs=[pl.BlockSpec((2, _RB, H), lambda i: (0, i, 0)),
                  pl.BlockSpec((2, _RB, H), lambda i: (0, i, 0)),
                  pl.BlockSpec((_RB, 1), lambda i: (i, 0)),
                  pl.BlockSpec((1, DH), lambda i: (0, 0)),
                  pl.BlockSpec((DH, DH), lambda i: (0, 0))],
        out_specs=pl.BlockSpec((2, _RB, H), lambda i: (0, i, 0)),
        out_shape=jax.ShapeDtypeStruct((2, N, H), jnp.float32),
    )(sm, t, dinv, b, W)


def _layer_last(sm, t, dinv, b, W):
    def body(s_ref, t_ref, d_ref, b_ref, w_ref, o_ref):
        dv = d_ref[...]
        hcat = jnp.concatenate([s_ref[0] + t_ref[0], s_ref[1] + t_ref[1]],
                               axis=1)
        h = jnp.maximum(hcat * dv + b_ref[...], 0.0)
        p = jnp.dot(h, w_ref[...], preferred_element_type=jnp.float32) * dv
        o_ref[...] = jnp.pad(p, ((0, 0), (0, H - DC)))

    return pl.pallas_call(
        body,
        grid=(N // _RB,),
        in_specs=[pl.BlockSpec((2, _RB, H), lambda i: (0, i, 0)),
                  pl.BlockSpec((2, _RB, H), lambda i: (0, i, 0)),
                  pl.BlockSpec((_RB, 1), lambda i: (i, 0)),
                  pl.BlockSpec((1, DH), lambda i: (0, 0)),
                  pl.BlockSpec((DH, DC), lambda i: (0, 0))],
        out_specs=pl.BlockSpec((_RB, H), lambda i: (i, 0)),
        out_shape=jax.ShapeDtypeStruct((N, H), jnp.float32),
    )(sm, t, dinv, b, W)


def _combine_out(s2, t2, dinv, bh):
    def body(s_ref, t_ref, d_ref, b_ref, o_ref):
        tot = s_ref[0, :, :DC] + s_ref[1, :, :DC] + t_ref[:, :DC]
        o_ref[...] = tot * d_ref[...] + b_ref[...]

    return pl.pallas_call(
        body,
        grid=(N // _RB,),
        in_specs=[pl.BlockSpec((2, _RB, H), lambda i: (0, i, 0)),
                  pl.BlockSpec((_RB, H), lambda i: (i, 0)),
                  pl.BlockSpec((_RB, 1), lambda i: (i, 0)),
                  pl.BlockSpec((1, DC), lambda i: (0, 0))],
        out_specs=pl.BlockSpec((_RB, DC), lambda i: (i, 0)),
        out_shape=jax.ShapeDtypeStruct((N, DC), jnp.float32),
    )(s2, t2, dinv, bh)


def kernel(x, edge_index, W0, b0, W1, b1, Wh, bh):
    src = edge_index[0]
    dst = edge_index[1]

    src2 = jnp.concatenate([src, src + N]).reshape(2, NS, NSUP_H, SUP, ECH)
    src_sp = src.reshape(2, NS, NSUP_S, SUP, ECH)
    ids = jnp.arange(N, dtype=jnp.int32)

    ones_c = jnp.ones((ECH, H), jnp.float32)
    zerosH = jnp.zeros((RCH, H), jnp.float32)

    degp = _deg_kernel(dst, ids, ones_c, zerosH)
    hw0 = _mm0(x, W0)
    dinv, t0 = _finish_t0(degp.reshape(2, N, H), hw0)
    t0f = t0.reshape(2 * N, H)
    s0 = _prop_half(t0f, src2, dst, ids, zerosH)
    t1 = _layer_mid(s0.reshape(2, N, H), t0, dinv, b0.reshape(1, DH), W1)
    t1f = t1.reshape(2 * N, H)
    s1 = _prop_half(t1f, src2, dst, ids, zerosH)
    t2 = _layer_last(s1.reshape(2, N, H), t1, dinv, b1.reshape(1, DH), Wh)
    s2 = _prop_full(t2, src_sp, dst, ids, zerosH)
    out = _combine_out(s2.reshape(2, N, H), t2, dinv, bh.reshape(1, DC))
    return out
